# baseline (speedup 1.0000x reference)
import functools

import jax
import jax.numpy as jnp
from jax import lax
from jax.experimental import pallas as pl
from jax.experimental.pallas import tpu as pltpu
from jax.experimental.pallas import tpu_sc as plsc

_B, _S, _D = 16, 2048, 512
_NC, _NS = 2, 16
_NW = _NC * _NS
_RPW = _B * _S // _NW
_C = 64
_VPR = _D // 16
_NCHUNK = _RPW // _C

_mesh = plsc.VectorSubcoreMesh(
    core_axis_name="c", subcore_axis_name="s", num_cores=_NC, num_subcores=_NS
)


@functools.partial(
    pl.kernel,
    out_type=jax.ShapeDtypeStruct((_B * _S * _D,), jnp.float32),
    mesh=_mesh,
    scratch_types=[
        pltpu.VMEM((32,), jnp.int32),
        pltpu.VMEM((_C * _D,), jnp.float32),
        pltpu.VMEM((_C * _D,), jnp.float32),
        pltpu.VMEM((_C * _D,), jnp.float32),
    ],
)
def _ape_sc(x_hbm, len_hbm, pos_hbm, out_hbm, len_v, xbuf, posbuf, zbuf):
    wid = lax.axis_index("s") * _NC + lax.axis_index("c")
    b = wid // 2
    half = wid % 2
    base_s = half * _RPW
    row0 = b * _S + base_s

    pltpu.sync_copy(len_hbm, len_v)
    seq_len = len_v[pl.ds(b, 16)][0]
    live = jnp.clip(seq_len - base_s, 0, _RPW)

    zero16 = jnp.zeros((16,), jnp.float32)

    def zinit(i, _):
        zbuf[pl.ds(i * 16, 16)] = zero16
        return 0

    lax.fori_loop(0, _C * _VPR, zinit, 0)

    def chunk(k, _):
        live_k = jnp.clip(live - k * _C, 0, _C)
        off = (row0 + k * _C) * _D

        @pl.when(live_k == 0)
        def _dead():
            pltpu.sync_copy(zbuf, out_hbm.at[pl.ds(off, _C * _D)])

        @pl.when(live_k > 0)
        def _live():
            pltpu.sync_copy(x_hbm.at[pl.ds(off, _C * _D)], xbuf)
            pltpu.sync_copy(
                pos_hbm.at[pl.ds((base_s + k * _C) * _D, _C * _D)], posbuf
            )

            def addv(i, _):
                plsc.addupdate(xbuf.at[pl.ds(i * 16, 16)], posbuf[pl.ds(i * 16, 16)])
                return 0

            lax.fori_loop(0, live_k * _VPR, addv, 0)

            def ztail(i, _):
                xbuf[pl.ds(i * 16, 16)] = zero16
                return 0

            lax.fori_loop(live_k * _VPR, _C * _VPR, ztail, 0)
            pltpu.sync_copy(xbuf, out_hbm.at[pl.ds(off, _C * _D)])

        return 0

    lax.fori_loop(0, _NCHUNK, chunk, 0)


@jax.jit
def kernel(x, seq_lengths, position_embeddings):
    lens_padded = jnp.concatenate(
        [seq_lengths.astype(jnp.int32), jnp.zeros((16,), jnp.int32)]
    )
    out = _ape_sc(
        x.reshape(-1), lens_padded, position_embeddings.reshape(-1)
    )
    return out.reshape(_B, _S, _D)


# speedup vs baseline: 1.9003x; 1.9003x over previous
import functools

import jax
import jax.numpy as jnp
from jax import lax
from jax.experimental import pallas as pl
from jax.experimental.pallas import tpu as pltpu
from jax.experimental.pallas import tpu_sc as plsc

_B, _S, _D = 16, 2048, 512
_NC, _NS = 2, 16
_NW = _NC * _NS
_C = 16
_CPB = _S // _C
_NCHUNK = _B * _S // _C
_J = _NCHUNK // _NW
_VPR = _D // 16
_CW = _C * _D
_NBUF = 4
_UNROLL = 8

_mesh = plsc.VectorSubcoreMesh(
    core_axis_name="c", subcore_axis_name="s", num_cores=_NC, num_subcores=_NS
)


@functools.partial(
    pl.kernel,
    out_type=jax.ShapeDtypeStruct((_B * _S * _D,), jnp.float32),
    mesh=_mesh,
    scratch_types=[
        pltpu.VMEM((32,), jnp.int32),
        [pltpu.VMEM((_CW,), jnp.float32)] * _NBUF,
        [pltpu.VMEM((_CW,), jnp.float32)] * _NBUF,
        pltpu.VMEM((_CW,), jnp.float32),
        [pltpu.SemaphoreType.DMA] * _NBUF,
        [pltpu.SemaphoreType.DMA] * _NBUF,
        [pltpu.SemaphoreType.DMA] * _NBUF,
    ],
)
def _ape_sc(x_hbm, len_hbm, pos_hbm, out_hbm, len_v, xb, pb, zb, sx, sp, so):
    wid = lax.axis_index("s") * _NC + lax.axis_index("c")

    pltpu.sync_copy(len_hbm, len_v)

    zero16 = jnp.zeros((16,), jnp.float32)

    def zinit(i, _):
        for u in range(_UNROLL):
            zb[pl.ds((i * _UNROLL + u) * 16, 16)] = zero16
        return 0

    lax.fori_loop(0, _CW // 16 // _UNROLL, zinit, 0)

    def chunk_info(j):
        c = j * _NW + wid
        b_idx = lax.shift_right_logical(c, 7)
        s0 = lax.bitwise_and(c, _CPB - 1) * _C
        seq_len = len_v[pl.ds(b_idx, 16)][0]
        live_k = jnp.clip(seq_len - s0, 0, _C)
        return c * _CW, s0 * _D, live_k

    def issue_in(j, b):
        off, poff, live_k = chunk_info(j)

        @pl.when(live_k > 0)
        def _():
            pltpu.make_async_copy(x_hbm.at[pl.ds(off, _CW)], xb[b], sx[b]).start()
            pltpu.make_async_copy(pos_hbm.at[pl.ds(poff, _CW)], pb[b], sp[b]).start()

    def wait_in(b):
        pltpu.make_async_copy(x_hbm.at[pl.ds(0, _CW)], xb[b], sx[b]).wait()
        pltpu.make_async_copy(pos_hbm.at[pl.ds(0, _CW)], pb[b], sp[b]).wait()

    def wait_out(b):
        pltpu.make_async_copy(zb, out_hbm.at[pl.ds(0, _CW)], so[b]).wait()

    def process(j, b):
        off, _poff, live_k = chunk_info(j)

        @pl.when(live_k > 0)
        def _live():
            wait_in(b)

            def addv(i, _):
                for u in range(_UNROLL):
                    idx = (i * _UNROLL + u) * 16
                    plsc.addupdate(xb[b].at[pl.ds(idx, 16)], pb[b][pl.ds(idx, 16)])
                return 0

            lax.fori_loop(0, live_k * (_VPR // _UNROLL), addv, 0)

            def ztail(i, _):
                for u in range(_UNROLL):
                    xb[b][pl.ds((i * _UNROLL + u) * 16, 16)] = zero16
                return 0

            lax.fori_loop(
                live_k * (_VPR // _UNROLL), _C * (_VPR // _UNROLL), ztail, 0
            )
            pltpu.make_async_copy(xb[b], out_hbm.at[pl.ds(off, _CW)], so[b]).start()

        @pl.when(live_k == 0)
        def _dead():
            pltpu.make_async_copy(zb, out_hbm.at[pl.ds(off, _CW)], so[b]).start()

    issue_in(0, 0)
    issue_in(1, 1)

    def step(g, _):
        for b in range(_NBUF):
            j = g * _NBUF + b
            t = j + 2
            bt = (b + 2) % _NBUF

            @pl.when(jnp.logical_and(t >= _NBUF, t < _J))
            def _drain():
                wait_out(bt)

            @pl.when(t < _J)
            def _pre():
                issue_in(t, bt)

            process(j, b)
        return 0

    lax.fori_loop(0, _J // _NBUF, step, 0)

    for b in range(_NBUF):
        wait_out(b)


@jax.jit
def kernel(x, seq_lengths, position_embeddings):
    lens_padded = jnp.concatenate(
        [seq_lengths.astype(jnp.int32), jnp.zeros((16,), jnp.int32)]
    )
    out = _ape_sc(
        x.reshape(-1), lens_padded, position_embeddings.reshape(-1)
    )
    return out.reshape(_B, _S, _D)


# speedup vs baseline: 4.8833x; 2.5697x over previous
import functools

import jax
import jax.numpy as jnp
from jax import lax
from jax.experimental import pallas as pl
from jax.experimental.pallas import tpu as pltpu
from jax.experimental.pallas import tpu_sc as plsc

_B, _S, _D = 16, 2048, 512
_NC, _NS = 2, 16
_NW = _NC * _NS
_C = 16
_CPB = _S // _C
_NCHUNK = _B * _S // _C
_J = _NCHUNK // _NW
_VPR = _D // 16
_NBUF = 4

_mesh = plsc.VectorSubcoreMesh(
    core_axis_name="c", subcore_axis_name="s", num_cores=_NC, num_subcores=_NS
)


@functools.partial(
    pl.kernel,
    out_type=jax.ShapeDtypeStruct((_B, _S, _D), jnp.float32),
    mesh=_mesh,
    scratch_types=[
        pltpu.VMEM((32,), jnp.int32),
        [pltpu.VMEM((_C, _D), jnp.float32)] * _NBUF,
        [pltpu.VMEM((_C, _D), jnp.float32)] * _NBUF,
        pltpu.VMEM((_C, _D), jnp.float32),
        [pltpu.SemaphoreType.DMA] * _NBUF,
        [pltpu.SemaphoreType.DMA] * _NBUF,
        [pltpu.SemaphoreType.DMA] * _NBUF,
    ],
)
def _ape_sc(x_hbm, len_hbm, pos_hbm, out_hbm, len_v, xb, pb, zb, sx, sp, so):
    wid = lax.axis_index("s") * _NC + lax.axis_index("c")

    pltpu.sync_copy(len_hbm, len_v.at[pl.ds(0, 16)])

    zero16 = jnp.zeros((16,), jnp.float32)

    def zinit(r, _):
        for u in range(_VPR):
            zb[r, pl.ds(u * 16, 16)] = zero16
        return 0

    lax.fori_loop(0, _C, zinit, 0)

    def chunk_info(j):
        c = j * _NW + wid
        b_idx = lax.shift_right_logical(c, 7)
        s0 = lax.bitwise_and(c, _CPB - 1) * _C
        seq_len = len_v[pl.ds(b_idx, 16)][0]
        live_k = jnp.clip(seq_len - s0, 0, _C)
        return b_idx, s0, live_k

    def issue_in(j, b):
        b_idx, s0, live_k = chunk_info(j)

        @pl.when(live_k > 0)
        def _():
            pltpu.make_async_copy(
                x_hbm.at[b_idx, pl.ds(s0, _C), :], xb[b], sx[b]
            ).start()
            pltpu.make_async_copy(
                pos_hbm.at[pl.ds(s0, _C), :], pb[b], sp[b]
            ).start()

    def wait_in(b):
        pltpu.make_async_copy(x_hbm.at[0, pl.ds(0, _C), :], xb[b], sx[b]).wait()
        pltpu.make_async_copy(pos_hbm.at[pl.ds(0, _C), :], pb[b], sp[b]).wait()

    def wait_out(b):
        pltpu.make_async_copy(zb, out_hbm.at[0, pl.ds(0, _C), :], so[b]).wait()

    def process(j, b):
        b_idx, s0, live_k = chunk_info(j)

        @pl.when(live_k > 0)
        def _live():
            wait_in(b)

            def addv(r, _):
                for u in range(_VPR):
                    plsc.addupdate(
                        xb[b].at[r, pl.ds(u * 16, 16)], pb[b][r, pl.ds(u * 16, 16)]
                    )
                return 0

            lax.fori_loop(0, live_k, addv, 0)

            def ztail(r, _):
                for u in range(_VPR):
                    xb[b][r, pl.ds(u * 16, 16)] = zero16
                return 0

            lax.fori_loop(live_k, _C, ztail, 0)
            pltpu.make_async_copy(
                xb[b], out_hbm.at[b_idx, pl.ds(s0, _C), :], so[b]
            ).start()

        @pl.when(live_k == 0)
        def _dead():
            pltpu.make_async_copy(
                zb, out_hbm.at[b_idx, pl.ds(s0, _C), :], so[b]
            ).start()

    issue_in(0, 0)
    issue_in(1, 1)

    def step(g, _):
        for b in range(_NBUF):
            j = g * _NBUF + b
            t = j + 2
            bt = (b + 2) % _NBUF

            @pl.when(jnp.logical_and(t >= _NBUF, t < _J))
            def _drain():
                wait_out(bt)

            @pl.when(t < _J)
            def _pre():
                issue_in(t, bt)

            process(j, b)
        return 0

    lax.fori_loop(0, _J // _NBUF, step, 0)

    for b in range(_NBUF):
        wait_out(b)


@jax.jit
def kernel(x, seq_lengths, position_embeddings):
    return _ape_sc(x, seq_lengths, position_embeddings)
